# baseline (speedup 1.0000x reference)
import jax
import jax.numpy as jnp
from jax import lax
from jax.experimental import pallas as pl
from jax.experimental.pallas import tpu as pltpu
from jax.experimental.pallas import tpu_sc as plsc

NUM_VOCAB = 100000
MAXLEN = 200
NUM_HID = 128
BATCH = 4096
SEQ = 200

NC, NS, L = 2, 16, 16
NW = NC * NS
ROWS = BATCH * SEQ
CHUNK = 128
CPW = ROWS // NW // CHUNK
NCHUNKS = ROWS // CHUNK
HGRP = NUM_HID // L
NITER = (CPW - 4) // 4


def _body(x2, s_emb, pos_emb, out, i0, i1, i2, i3, b0, b1, b2, b3, posb,
          gi0, gi1, gi2, gi3, g0, g1, g2, g3, o0, o1, o2, o3):
    wid = lax.axis_index("s") * NC + lax.axis_index("c")
    cb = wid * CPW

    pltpu.sync_copy(pos_emb, posb)

    idxs = (i0, i1, i2, i3)
    bufs = (b0, b1, b2, b3)
    isems = (gi0, gi1, gi2, gi3)
    gsems = (g0, g1, g2, g3)
    osems = (o0, o1, o2, o3)

    def start_idx(c, p):
        cc = jnp.minimum(cb + c, NCHUNKS - 1)
        pltpu.async_copy(x2.at[pl.ds(cc, 1)], idxs[p], isems[p])

    def wait_idx(p):
        pltpu.make_async_copy(x2.at[pl.ds(0, 1)], idxs[p], isems[p]).wait()

    def start_gather(c, p):
        pltpu.async_copy(s_emb.at[idxs[p].at[0]], bufs[p], gsems[p])

    def wait_gather(p):
        pltpu.make_async_copy(s_emb.at[idxs[p].at[0]], bufs[p], gsems[p]).wait()

    def start_scatter(c, p):
        pltpu.async_copy(bufs[p], out.at[pl.ds((cb + c) * CHUNK, CHUNK)], osems[p])

    def wait_scatter(p):
        pltpu.make_async_copy(bufs[p], out.at[pl.ds(0, CHUNK)], osems[p]).wait()

    def add_pos(c, p):
        buf = bufs[p]
        off = lax.rem((cb + c) * CHUNK, MAXLEN)

        def add_row(r, pr):
            for cg in range(HGRP):
                pv = posb[pr, pl.ds(cg * L, L)]
                plsc.addupdate(buf.at[r, pl.ds(cg * L, L)], pv)
            nxt = pr + 1
            return lax.select(nxt == MAXLEN, 0, nxt)

        lax.fori_loop(0, CHUNK, add_row, off, unroll=2)

    for c in range(4):
        start_idx(c, c)
    for c in range(2):
        wait_idx(c)
        start_gather(c, c)

    def step(it, carry):
        c0 = 4 * it
        for j in range(4):
            p = j
            pn = (j + 2) % 4
            c = c0 + j
            wait_gather(p)
            start_idx(c + 4, p)
            add_pos(c, p)
            start_scatter(c, p)
            wait_idx(pn)
            if j < 2:
                @pl.when(it > 0)
                def _():
                    wait_scatter(pn)
            else:
                wait_scatter(pn)
            start_gather(c + 2, pn)

        return carry

    lax.fori_loop(0, NITER, step, 0)

    for c, p in ((CPW - 4, 0), (CPW - 3, 1)):
        wait_gather(p)
        add_pos(c, p)
        start_scatter(c, p)
        pn = p + 2
        wait_idx(pn)
        wait_scatter(pn)
        start_gather(c + 2, pn)
    for c, p in ((CPW - 2, 2), (CPW - 1, 3)):
        wait_gather(p)
        add_pos(c, p)
        start_scatter(c, p)
    for p in range(4):
        wait_scatter(p)


@jax.jit
def _run(x2, s_emb, pos_emb):
    mesh = plsc.VectorSubcoreMesh(core_axis_name="c", subcore_axis_name="s")
    return pl.kernel(
        _body,
        out_type=jax.ShapeDtypeStruct((ROWS, NUM_HID), jnp.float32),
        mesh=mesh,
        scratch_types=[
            pltpu.VMEM((1, CHUNK), jnp.int32),
            pltpu.VMEM((1, CHUNK), jnp.int32),
            pltpu.VMEM((1, CHUNK), jnp.int32),
            pltpu.VMEM((1, CHUNK), jnp.int32),
            pltpu.VMEM((CHUNK, NUM_HID), jnp.float32),
            pltpu.VMEM((CHUNK, NUM_HID), jnp.float32),
            pltpu.VMEM((CHUNK, NUM_HID), jnp.float32),
            pltpu.VMEM((CHUNK, NUM_HID), jnp.float32),
            pltpu.VMEM((MAXLEN, NUM_HID), jnp.float32),
            pltpu.SemaphoreType.DMA,
            pltpu.SemaphoreType.DMA,
            pltpu.SemaphoreType.DMA,
            pltpu.SemaphoreType.DMA,
            pltpu.SemaphoreType.DMA,
            pltpu.SemaphoreType.DMA,
            pltpu.SemaphoreType.DMA,
            pltpu.SemaphoreType.DMA,
            pltpu.SemaphoreType.DMA,
            pltpu.SemaphoreType.DMA,
            pltpu.SemaphoreType.DMA,
            pltpu.SemaphoreType.DMA,
        ],
    )(x2, s_emb, pos_emb)


def kernel(x, s_emb, pos_emb):
    x2 = x.astype(jnp.int32).reshape(ROWS // CHUNK, CHUNK)
    out = _run(x2, s_emb, pos_emb)
    return out.reshape(BATCH, SEQ, NUM_HID)
